# baseline (speedup 1.0000x reference)
import jax
import jax.numpy as jnp
from jax import lax
from jax.experimental import pallas as pl
from jax.experimental.pallas import tpu as pltpu
from jax.experimental.pallas import tpu_sc as plsc

N = 50000
HID = 128
H2 = HID // 2
E = 400000

NC = 2
NS = 16
LANES = 16

CHUNK = 12544
NCHUNK = 4
NPAD = CHUNK * NCHUNK
ACC_ROWS = 12672
DUMMY = CHUNK
EPT = 25008
EPAD = NS * EPT
EBLK = 1024
NBLK = 25
B = 96

NCPAD = 50176
NCST = NCPAD // NS
CDUMMY = N


def _sc_compiler_params():
    return pltpu.CompilerParams(needs_layout_passes=False)


def _core_ids():
    return lax.axis_index("c"), lax.axis_index("s")


def _seg_body(table, srcp, dstp, srcn, dstn, out_p, out_n,
              acc, ebs, ebd, csrc0, cdst0, stage0, csrc1, cdst1, stage1,
              sem0, sem1):
    cid, sid = _core_ids()
    zvec = jnp.zeros((LANES,), jnp.float32)
    slots = ((csrc0, cdst0, stage0, sem0), (csrc1, cdst1, stage1, sem1))

    def prefill(cs, cd):
        for k in range(B // LANES):
            cs[pl.ds(k * LANES, LANES)] = jnp.zeros((LANES,), jnp.int32)
            cd[0, pl.ds(k * LANES, LANES)] = jnp.full((LANES,), DUMMY, jnp.int32)

    def flush(j):
        cs, cd, stg, sem = slots[j]
        ocs, ocd, ostg, osem = slots[1 - j]
        pltpu.sync_copy(table.at[cs], stg)
        pltpu.async_copy(stg, acc.at[cd.at[0]], sem, add=True)
        pltpu.make_async_copy(ostg, acc.at[ocd.at[0]], osem).wait()
        prefill(ocs, ocd)

    for r in range(2 * NC):
        sign = r // 2
        lc = r % 2
        base = (cid * 2 + lc) * CHUNK
        s_hbm = srcp if sign == 0 else srcn
        d_hbm = dstp if sign == 0 else dstn

        def zf(i, _):
            stage0[i // 8, pl.ds((i % 8) * LANES, LANES)] = zvec
            stage1[i // 8, pl.ds((i % 8) * LANES, LANES)] = zvec
            return 0
        lax.fori_loop(0, B * (HID // LANES), zf, 0)
        zst = ACC_ROWS // NS
        for k in range(zst // B):
            pltpu.sync_copy(stage0, acc.at[pl.ds(sid * zst + k * B, B)])
        if zst % B:
            pltpu.sync_copy(stage0, acc.at[pl.ds(sid * zst + zst - B, B)])
        for cs, cd, _stg, _sem in slots:
            prefill(cs, cd)
        plsc.subcore_barrier()
        pltpu.async_copy(stage1, acc.at[cdst1.at[0]], sem1, add=True)

        def step(i, carry):
            w, sl = carry
            full = w > (B - LANES)
            for j in range(2):
                @pl.when(full & (sl == j))
                def _(j=j):
                    flush(j)
            sl = jnp.where(full, 1 - sl, sl)
            w = jnp.where(full, 0, w)
            d = ebd[pl.ds(i * LANES, LANES)]
            m = (d >= base) & (d < base + CHUNK)
            s = ebs[pl.ds(i * LANES, LANES)]
            for j, (cs, cd, _stg, _sem) in enumerate(slots):
                @pl.when(sl == j)
                def _(cs=cs, cd=cd):
                    plsc.store_compressed(cs.at[pl.ds(w, LANES)], s, mask=m)
                    plsc.store_compressed(cd.at[0, pl.ds(w, LANES)], d - base, mask=m)
            return (w + jnp.sum(m.astype(jnp.int32)), sl)

        def blockloop(blk, carry):
            off = sid * EPT + blk * EBLK
            pltpu.sync_copy(s_hbm.at[pl.ds(off, EBLK)], ebs)
            pltpu.sync_copy(d_hbm.at[pl.ds(off, EBLK)], ebd)
            return lax.fori_loop(0, EBLK // LANES, step, carry)

        wp, slot = lax.fori_loop(0, NBLK - 1, blockloop,
                                 (jnp.int32(0), jnp.int32(0)))
        tail = EPT - (NBLK - 1) * EBLK
        toff = sid * EPT + (NBLK - 1) * EBLK
        pltpu.sync_copy(s_hbm.at[pl.ds(toff, tail)], ebs.at[pl.ds(0, tail)])
        pltpu.sync_copy(d_hbm.at[pl.ds(toff, tail)], ebd.at[pl.ds(0, tail)])
        wp, slot = lax.fori_loop(0, tail // LANES, step, (wp, slot))

        for j in range(2):
            @pl.when((wp > 0) & (slot == j))
            def _(j=j):
                flush(j)
        slot = jnp.where(wp > 0, 1 - slot, slot)
        for j in range(2):
            @pl.when(slot == j)
            def _(j=j):
                _ocs, ocd, ostg, osem = slots[1 - j]
                pltpu.make_async_copy(ostg, acc.at[ocd.at[0]], osem).wait()
        plsc.subcore_barrier()

        o = out_p if sign == 0 else out_n
        st = CHUNK // NS
        offs = [k * B for k in range(st // B)]
        if st % B:
            offs.append(st - B)
        for k in offs:
            pltpu.sync_copy(acc.at[pl.ds(sid * st + k, B)], stage0)
            pltpu.sync_copy(stage0, o.at[pl.ds(base + sid * st + k, B)])
        plsc.subcore_barrier()


def _make_seg_kernel():
    outs = (jax.ShapeDtypeStruct((NPAD, HID), jnp.float32),
            jax.ShapeDtypeStruct((NPAD, HID), jnp.float32))
    scratch = [
        pltpu.VMEM_SHARED((ACC_ROWS, HID), jnp.float32),
        pltpu.VMEM((EBLK,), jnp.int32),
        pltpu.VMEM((EBLK,), jnp.int32),
        pltpu.VMEM((B,), jnp.int32),
        pltpu.VMEM((1, B), jnp.int32),
        pltpu.VMEM((B, HID), jnp.float32),
        pltpu.VMEM((B,), jnp.int32),
        pltpu.VMEM((1, B), jnp.int32),
        pltpu.VMEM((B, HID), jnp.float32),
        pltpu.SemaphoreType.DMA,
        pltpu.SemaphoreType.DMA,
    ]
    mesh = plsc.VectorSubcoreMesh(core_axis_name="c", subcore_axis_name="s")
    return pl.kernel(_seg_body, out_type=outs, mesh=mesh,
                     compiler_params=_sc_compiler_params(),
                     scratch_types=scratch)


def _cnt_body(dst2, inv2, hist, ebd, res, sh):
    cid, sid = _core_ids()
    ones = jnp.full((LANES,), 1.0, jnp.float32)

    def z(i, _):
        hist[pl.ds(i * LANES, LANES)] = jnp.zeros((LANES,), jnp.float32)
        return 0
    lax.fori_loop(0, NCPAD // LANES, z, 0)

    for blk in range(NBLK):
        blen = EBLK if blk < NBLK - 1 else EPT - (NBLK - 1) * EBLK
        off = cid * EPAD + sid * EPT + blk * EBLK
        pltpu.sync_copy(dst2.at[pl.ds(off, blen)], ebd.at[pl.ds(0, blen)])

        def vec(i, _):
            d = ebd[pl.ds(i * LANES, LANES)]
            dz = jnp.where(d >= 0, d, CDUMMY)
            plsc.addupdate_scatter(hist, [dz], ones)
            return 0
        lax.fori_loop(0, blen // LANES, vec, 0)

    _cnt_reduce(sh, cid, sid, hist, res, inv2)


def _cnt_reduce(sh, cid, sid, hist, res, inv2):
    pltpu.sync_copy(hist, sh.at[pl.ds(sid * NCPAD, NCPAD)])
    plsc.subcore_barrier()
    for h in range(NS):
        pltpu.sync_copy(sh.at[pl.ds(h * NCPAD + sid * NCST, NCST)],
                        hist.at[pl.ds(h * NCST, NCST)])

    def red(j, _):
        v = jnp.zeros((LANES,), jnp.float32)
        for h in range(NS):
            v = v + hist[pl.ds(h * NCST + j * LANES, LANES)]
        res[pl.ds(j * LANES, LANES)] = 1.0 / jnp.maximum(v, 1.0)
        return 0
    lax.fori_loop(0, NCST // LANES, red, 0)

    pltpu.sync_copy(res, inv2.at[pl.ds(cid * NCPAD + sid * NCST, NCST)])


def _make_cnt_kernel():
    mesh = plsc.VectorSubcoreMesh(core_axis_name="c", subcore_axis_name="s")
    return pl.kernel(
        _cnt_body,
        out_type=jax.ShapeDtypeStruct((NC * NCPAD,), jnp.float32),
        mesh=mesh,
        compiler_params=_sc_compiler_params(),
        scratch_types=[
            pltpu.VMEM((NCPAD,), jnp.float32),
            pltpu.VMEM((EBLK,), jnp.int32),
            pltpu.VMEM((NCST,), jnp.float32),
            pltpu.VMEM_SHARED((NS * NCPAD,), jnp.float32),
        ])


BN = 400


def _l1_body(x_ref, sp_ref, sn_ref, cp_ref, cn_ref,
             wpl, wpr, bp, wnl, wnr, bneg, o_ref):
    x = x_ref[...]
    hp = (jnp.dot(sp_ref[...] * cp_ref[:, 0:1], wpl[...], preferred_element_type=jnp.float32)
          + jnp.dot(x, wpr[...], preferred_element_type=jnp.float32) + bp[...])
    hn = (jnp.dot(sn_ref[...] * cn_ref[:, 0:1], wnl[...], preferred_element_type=jnp.float32)
          + jnp.dot(x, wnr[...], preferred_element_type=jnp.float32) + bneg[...])
    o_ref[...] = jnp.maximum(jnp.concatenate([hp, hn], axis=1), 0.0)


def _l2_body(z_ref, sp_ref, sn_ref, cp_ref, cn_ref,
             wpl, wpr, bp, wnl, wnr, bneg, o_ref):
    P = sp_ref[...] * cp_ref[:, 0:1]
    Q = sn_ref[...] * cn_ref[:, 0:1]
    z = z_ref[...]
    catp = jnp.concatenate([P[:, :H2], Q[:, H2:]], axis=1)
    catn = jnp.concatenate([P[:, H2:], Q[:, :H2]], axis=1)
    hp = (jnp.dot(catp, wpl[...], preferred_element_type=jnp.float32)
          + jnp.dot(z[:, :H2], wpr[...], preferred_element_type=jnp.float32) + bp[...])
    hn = (jnp.dot(catn, wnl[...], preferred_element_type=jnp.float32)
          + jnp.dot(z[:, H2:], wnr[...], preferred_element_type=jnp.float32) + bneg[...])
    o_ref[...] = jnp.maximum(jnp.concatenate([hp, hn], axis=1), 0.0)


def _dense_call(body, x, sp, sn, cp, cn, wl, wr, b, wl2, wr2, b2):
    row = lambda i: (i, 0)
    zero = lambda i: (0, 0)
    specs = [
        pl.BlockSpec((BN, HID), row),
        pl.BlockSpec((BN, HID), row),
        pl.BlockSpec((BN, HID), row),
        pl.BlockSpec((BN, LANES), row),
        pl.BlockSpec((BN, LANES), row),
        pl.BlockSpec(wl.shape, zero),
        pl.BlockSpec(wr.shape, zero),
        pl.BlockSpec((1, H2), zero),
        pl.BlockSpec(wl2.shape, zero),
        pl.BlockSpec(wr2.shape, zero),
        pl.BlockSpec((1, H2), zero),
    ]
    return pl.pallas_call(
        body,
        grid=(N // BN,),
        in_specs=specs,
        out_specs=pl.BlockSpec((BN, HID), row),
        out_shape=jax.ShapeDtypeStruct((N, HID), jnp.float32),
    )(x, sp, sn, cp, cn, wl, wr, b.reshape(1, H2), wl2, wr2, b2.reshape(1, H2))


def kernel(pos_edge_index, neg_edge_index, users_emb, items_emb,
           W1_pos_l, W1_pos_r, b1_pos, W1_neg_l, W1_neg_r, b1_neg,
           W2_pos_l, W2_pos_r, b2_pos, W2_neg_l, W2_neg_r, b2_neg):
    x = jnp.concatenate([users_emb, items_emb], axis=0)
    pad = EPAD - E
    i32 = jnp.int32
    srcp = jnp.concatenate([pos_edge_index[0].astype(i32), jnp.zeros((pad,), i32)])
    dstp = jnp.concatenate([pos_edge_index[1].astype(i32), jnp.full((pad,), -1, i32)])
    srcn = jnp.concatenate([neg_edge_index[0].astype(i32), jnp.zeros((pad,), i32)])
    dstn = jnp.concatenate([neg_edge_index[1].astype(i32), jnp.full((pad,), -1, i32)])

    inv2 = _make_cnt_kernel()(jnp.concatenate([dstp, dstn]))
    inv_p = jnp.broadcast_to(inv2[:NCPAD].reshape(NCPAD, 1), (NCPAD, LANES))
    inv_n = jnp.broadcast_to(inv2[NCPAD:].reshape(NCPAD, 1), (NCPAD, LANES))
    seg = _make_seg_kernel()
    sum_p, sum_n = seg(x, srcp, dstp, srcn, dstn)
    z1 = _dense_call(_l1_body, x, sum_p, sum_n, inv_p, inv_n,
                     W1_pos_l, W1_pos_r, b1_pos, W1_neg_l, W1_neg_r, b1_neg)
    s2p, s2n = seg(z1, srcp, dstp, srcn, dstn)
    z2 = _dense_call(_l2_body, z1, s2p, s2n, inv_p, inv_n,
                     W2_pos_l, W2_pos_r, b2_pos, W2_neg_l, W2_neg_r, b2_neg)
    return z2


# speedup vs baseline: 1.2955x; 1.2955x over previous
import jax
import jax.numpy as jnp
from jax import lax
from jax.experimental import pallas as pl
from jax.experimental.pallas import tpu as pltpu
from jax.experimental.pallas import tpu_sc as plsc

N = 50000
HID = 128
H2 = HID // 2
E = 400000

NC = 2
NS = 16
LANES = 16

CHUNK = 12544
NCHUNK = 4
NPAD = CHUNK * NCHUNK
ACC_ROWS = 12800
DUMMY = CHUNK
EPT = 25008
EPAD = NS * EPT
EBLK = 2048
NBLK = 13
B = 128

NCPAD = 50176
NCST = NCPAD // NS
CDUMMY = N


def _sc_compiler_params():
    return pltpu.CompilerParams(needs_layout_passes=False)


def _core_ids():
    return lax.axis_index("c"), lax.axis_index("s")


def _seg_body(table, srcp, dstp, srcn, dstn, out_p, out_n,
              acc, ebs, ebd, csrc, cdst, cshadow, stage, sem):
    cid, sid = _core_ids()
    zvec = jnp.zeros((LANES,), jnp.float32)

    def prefill():
        for k in range(B // LANES):
            csrc[pl.ds(k * LANES, LANES)] = jnp.zeros((LANES,), jnp.int32)
            cdst[0, pl.ds(k * LANES, LANES)] = jnp.full((LANES,), DUMMY, jnp.int32)

    def flush():
        pltpu.make_async_copy(stage, acc.at[cshadow.at[0]], sem).wait()
        for k in range(B // LANES):
            cshadow[0, pl.ds(k * LANES, LANES)] = cdst[0, pl.ds(k * LANES, LANES)]
        pltpu.sync_copy(table.at[csrc], stage)
        pltpu.async_copy(stage, acc.at[cshadow.at[0]], sem, add=True)

    for r in range(2 * NC):
        sign = r // 2
        lc = r % 2
        base = (cid * 2 + lc) * CHUNK
        s_hbm = srcp if sign == 0 else srcn
        d_hbm = dstp if sign == 0 else dstn

        def zf(i, _):
            stage[i // 8, pl.ds((i % 8) * LANES, LANES)] = zvec
            return 0
        lax.fori_loop(0, B * (HID // LANES), zf, 0)
        zst = ACC_ROWS // NS
        for k in range(zst // B):
            pltpu.sync_copy(stage, acc.at[pl.ds(sid * zst + k * B, B)])
        if zst % B:
            pltpu.sync_copy(stage, acc.at[pl.ds(sid * zst + zst - B, B)])
        prefill()
        for k in range(B // LANES):
            cshadow[0, pl.ds(k * LANES, LANES)] = jnp.full((LANES,), DUMMY, jnp.int32)
        plsc.subcore_barrier()
        pltpu.async_copy(stage, acc.at[cshadow.at[0]], sem, add=True)

        wp = jnp.int32(0)
        for blk in range(NBLK):
            blen = EBLK if blk < NBLK - 1 else EPT - (NBLK - 1) * EBLK
            off = sid * EPT + blk * EBLK
            pltpu.sync_copy(s_hbm.at[pl.ds(off, blen)], ebs.at[pl.ds(0, blen)])
            pltpu.sync_copy(d_hbm.at[pl.ds(off, blen)], ebd.at[pl.ds(0, blen)])

            def step(i, w):
                full = w > (B - LANES)
                @pl.when(full)
                def _():
                    flush()
                    prefill()
                w = jnp.where(full, 0, w)
                d = ebd[pl.ds(i * LANES, LANES)]
                m = (d >= base) & (d < base + CHUNK)
                s = ebs[pl.ds(i * LANES, LANES)]
                plsc.store_compressed(csrc.at[pl.ds(w, LANES)], s, mask=m)
                plsc.store_compressed(cdst.at[0, pl.ds(w, LANES)], d - base, mask=m)
                return w + jnp.sum(m.astype(jnp.int32))

            wp = lax.fori_loop(0, blen // LANES, step, wp)

        @pl.when(wp > 0)
        def _():
            flush()
            prefill()
        pltpu.make_async_copy(stage, acc.at[cshadow.at[0]], sem).wait()
        plsc.subcore_barrier()

        o = out_p if sign == 0 else out_n
        st = CHUNK // NS
        offs = [k * B for k in range(st // B)]
        if st % B:
            offs.append(st - B)
        for k in offs:
            pltpu.sync_copy(acc.at[pl.ds(sid * st + k, B)], stage)
            pltpu.sync_copy(stage, o.at[pl.ds(base + sid * st + k, B)])
        plsc.subcore_barrier()


def _make_seg_kernel():
    outs = (jax.ShapeDtypeStruct((NPAD, HID), jnp.float32),
            jax.ShapeDtypeStruct((NPAD, HID), jnp.float32))
    scratch = [
        pltpu.VMEM_SHARED((ACC_ROWS, HID), jnp.float32),
        pltpu.VMEM((EBLK,), jnp.int32),
        pltpu.VMEM((EBLK,), jnp.int32),
        pltpu.VMEM((B,), jnp.int32),
        pltpu.VMEM((1, B), jnp.int32),
        pltpu.VMEM((1, B), jnp.int32),
        pltpu.VMEM((B, HID), jnp.float32),
        pltpu.SemaphoreType.DMA,
    ]
    mesh = plsc.VectorSubcoreMesh(core_axis_name="c", subcore_axis_name="s")
    return pl.kernel(_seg_body, out_type=outs, mesh=mesh,
                     compiler_params=_sc_compiler_params(),
                     scratch_types=scratch)


def _cnt_body(dst2, inv2, hist, ebd, res, sh):
    cid, sid = _core_ids()
    ones = jnp.full((LANES,), 1.0, jnp.float32)

    def z(i, _):
        hist[pl.ds(i * LANES, LANES)] = jnp.zeros((LANES,), jnp.float32)
        return 0
    lax.fori_loop(0, NCPAD // LANES, z, 0)

    for blk in range(NBLK):
        blen = EBLK if blk < NBLK - 1 else EPT - (NBLK - 1) * EBLK
        off = cid * EPAD + sid * EPT + blk * EBLK
        pltpu.sync_copy(dst2.at[pl.ds(off, blen)], ebd.at[pl.ds(0, blen)])

        def vec(i, _):
            d = ebd[pl.ds(i * LANES, LANES)]
            dz = jnp.where(d >= 0, d, CDUMMY)
            plsc.addupdate_scatter(hist, [dz], ones)
            return 0
        lax.fori_loop(0, blen // LANES, vec, 0)

    _cnt_reduce(sh, cid, sid, hist, res, inv2)


def _cnt_reduce(sh, cid, sid, hist, res, inv2):
    pltpu.sync_copy(hist, sh.at[pl.ds(sid * NCPAD, NCPAD)])
    plsc.subcore_barrier()
    for h in range(NS):
        pltpu.sync_copy(sh.at[pl.ds(h * NCPAD + sid * NCST, NCST)],
                        hist.at[pl.ds(h * NCST, NCST)])

    def red(j, _):
        v = jnp.zeros((LANES,), jnp.float32)
        for h in range(NS):
            v = v + hist[pl.ds(h * NCST + j * LANES, LANES)]
        res[pl.ds(j * LANES, LANES)] = 1.0 / jnp.maximum(v, 1.0)
        return 0
    lax.fori_loop(0, NCST // LANES, red, 0)

    pltpu.sync_copy(res, inv2.at[pl.ds(cid * NCPAD + sid * NCST, NCST)])


def _make_cnt_kernel():
    mesh = plsc.VectorSubcoreMesh(core_axis_name="c", subcore_axis_name="s")
    return pl.kernel(
        _cnt_body,
        out_type=jax.ShapeDtypeStruct((NC * NCPAD,), jnp.float32),
        mesh=mesh,
        compiler_params=_sc_compiler_params(),
        scratch_types=[
            pltpu.VMEM((NCPAD,), jnp.float32),
            pltpu.VMEM((EBLK,), jnp.int32),
            pltpu.VMEM((NCST,), jnp.float32),
            pltpu.VMEM_SHARED((NS * NCPAD,), jnp.float32),
        ])


BN = 400


def _l1_body(x_ref, sp_ref, sn_ref, cp_ref, cn_ref,
             wpl, wpr, bp, wnl, wnr, bneg, o_ref):
    x = x_ref[...]
    hp = (jnp.dot(sp_ref[...] * cp_ref[:, 0:1], wpl[...], preferred_element_type=jnp.float32)
          + jnp.dot(x, wpr[...], preferred_element_type=jnp.float32) + bp[...])
    hn = (jnp.dot(sn_ref[...] * cn_ref[:, 0:1], wnl[...], preferred_element_type=jnp.float32)
          + jnp.dot(x, wnr[...], preferred_element_type=jnp.float32) + bneg[...])
    o_ref[...] = jnp.maximum(jnp.concatenate([hp, hn], axis=1), 0.0)


def _l2_body(z_ref, sp_ref, sn_ref, cp_ref, cn_ref,
             wpl, wpr, bp, wnl, wnr, bneg, o_ref):
    P = sp_ref[...] * cp_ref[:, 0:1]
    Q = sn_ref[...] * cn_ref[:, 0:1]
    z = z_ref[...]
    catp = jnp.concatenate([P[:, :H2], Q[:, H2:]], axis=1)
    catn = jnp.concatenate([P[:, H2:], Q[:, :H2]], axis=1)
    hp = (jnp.dot(catp, wpl[...], preferred_element_type=jnp.float32)
          + jnp.dot(z[:, :H2], wpr[...], preferred_element_type=jnp.float32) + bp[...])
    hn = (jnp.dot(catn, wnl[...], preferred_element_type=jnp.float32)
          + jnp.dot(z[:, H2:], wnr[...], preferred_element_type=jnp.float32) + bneg[...])
    o_ref[...] = jnp.maximum(jnp.concatenate([hp, hn], axis=1), 0.0)


def _dense_call(body, x, sp, sn, cp, cn, wl, wr, b, wl2, wr2, b2):
    row = lambda i: (i, 0)
    zero = lambda i: (0, 0)
    specs = [
        pl.BlockSpec((BN, HID), row),
        pl.BlockSpec((BN, HID), row),
        pl.BlockSpec((BN, HID), row),
        pl.BlockSpec((BN, LANES), row),
        pl.BlockSpec((BN, LANES), row),
        pl.BlockSpec(wl.shape, zero),
        pl.BlockSpec(wr.shape, zero),
        pl.BlockSpec((1, H2), zero),
        pl.BlockSpec(wl2.shape, zero),
        pl.BlockSpec(wr2.shape, zero),
        pl.BlockSpec((1, H2), zero),
    ]
    return pl.pallas_call(
        body,
        grid=(N // BN,),
        in_specs=specs,
        out_specs=pl.BlockSpec((BN, HID), row),
        out_shape=jax.ShapeDtypeStruct((N, HID), jnp.float32),
    )(x, sp, sn, cp, cn, wl, wr, b.reshape(1, H2), wl2, wr2, b2.reshape(1, H2))


def kernel(pos_edge_index, neg_edge_index, users_emb, items_emb,
           W1_pos_l, W1_pos_r, b1_pos, W1_neg_l, W1_neg_r, b1_neg,
           W2_pos_l, W2_pos_r, b2_pos, W2_neg_l, W2_neg_r, b2_neg):
    x = jnp.concatenate([users_emb, items_emb], axis=0)
    pad = EPAD - E
    i32 = jnp.int32
    srcp = jnp.concatenate([pos_edge_index[0].astype(i32), jnp.zeros((pad,), i32)])
    dstp = jnp.concatenate([pos_edge_index[1].astype(i32), jnp.full((pad,), -1, i32)])
    srcn = jnp.concatenate([neg_edge_index[0].astype(i32), jnp.zeros((pad,), i32)])
    dstn = jnp.concatenate([neg_edge_index[1].astype(i32), jnp.full((pad,), -1, i32)])

    inv2 = _make_cnt_kernel()(jnp.concatenate([dstp, dstn]))
    inv_p = jnp.broadcast_to(inv2[:NCPAD].reshape(NCPAD, 1), (NCPAD, LANES))
    inv_n = jnp.broadcast_to(inv2[NCPAD:].reshape(NCPAD, 1), (NCPAD, LANES))
    seg = _make_seg_kernel()
    sum_p, sum_n = seg(x, srcp, dstp, srcn, dstn)
    z1 = _dense_call(_l1_body, x, sum_p, sum_n, inv_p, inv_n,
                     W1_pos_l, W1_pos_r, b1_pos, W1_neg_l, W1_neg_r, b1_neg)
    s2p, s2n = seg(z1, srcp, dstp, srcn, dstn)
    z2 = _dense_call(_l2_body, z1, s2p, s2n, inv_p, inv_n,
                     W2_pos_l, W2_pos_r, b2_pos, W2_neg_l, W2_neg_r, b2_neg)
    return z2
